# baseline (speedup 1.0000x reference)
import functools

import jax
import jax.numpy as jnp
from jax import lax
from jax.experimental import pallas as pl
from jax.experimental.pallas import tpu as pltpu
from jax.experimental.pallas import tpu_sc as plsc

NC = 2
NS = 16
NW = NC * NS


K = 4
R = 64
CH = K * R
SBR = 16


def _emit_table_loop(table_hbm, idx_hbm, out_hbm, idx_w_base, out_w_base,
                     n_super, idx_v, rows_v, gsem, osem):
    n_chunks = n_super * 4

    def wait_out(b):
        pltpu.make_async_copy(rows_v[b], out_hbm.at[pl.ds(0, CH)], osem[b]).wait()

    def wait_gathers(b):
        pltpu.make_async_copy(out_hbm.at[pl.ds(0, CH)], rows_v[b], gsem[b]).wait()

    @pl.loop(0, n_super // 2)
    def _(gp):
        for ib in (0, 1):
            sb = gp * 2 + ib
            pltpu.sync_copy(idx_hbm.at[pl.ds(idx_w_base + sb * SBR, SBR)],
                            idx_v[ib])
            for c in range(4):
                b = c & 1
                if c >= 2:
                    wait_out(b)
                else:
                    @pl.when(sb >= 1)
                    def _w():
                        wait_out(b)
                for j in range(K):
                    pltpu.async_copy(
                        table_hbm.at[idx_v[ib].at[c * K + j]],
                        rows_v[b].at[pl.ds(j * R, R)],
                        gsem[b])
                prev_out = out_w_base + (sb * 4 + c - 1) * CH

                def _drain(prev_out=prev_out, b=b):
                    wait_gathers(1 - b)
                    pltpu.async_copy(rows_v[1 - b],
                                     out_hbm.at[pl.ds(prev_out, CH)],
                                     osem[1 - b])
                if c >= 1:
                    _drain()
                else:
                    @pl.when(sb >= 1)
                    def _d():
                        _drain()
    wait_gathers(1)
    pltpu.async_copy(
        rows_v[1],
        out_hbm.at[pl.ds(out_w_base + (n_chunks - 1) * CH, CH)], osem[1])
    wait_out(0)
    wait_out(1)


def _emit_pooled_loop(table_hbm, idx_hbm, out_hbm, wid, idx_v, rows_v,
                      pooled_v, gsem, osem, isem, n_super, s_len):
    IDXR = 40
    CKI = 5
    CROWS = CKI * R
    OROWS = CROWS // s_len
    per_w_idx = n_super * IDXR
    idx_base = wid * per_w_idx
    out_base = wid * (per_w_idx * R // s_len)

    def fire_chunk(ibuf, c, b):
        for j in range(CKI):
            pltpu.async_copy(
                table_hbm.at[idx_v[ibuf].at[c * CKI + j]],
                rows_v[b].at[pl.ds(j * R, R)],
                gsem[b])

    def wait_gathers(b):
        pltpu.make_async_copy(table_hbm.at[pl.ds(0, CROWS)], rows_v[b],
                              gsem[b]).wait()

    def wait_out(pb):
        pltpu.make_async_copy(pooled_v[pb], out_hbm.at[pl.ds(0, OROWS)],
                              osem[pb]).wait()

    def load_idx_sync(sb, ibuf):
        pltpu.sync_copy(idx_hbm.at[pl.ds(idx_base + sb * IDXR, IDXR)],
                        idx_v[ibuf])

    def load_idx_async(sb, ibuf):
        pltpu.async_copy(idx_hbm.at[pl.ds(idx_base + sb * IDXR, IDXR)],
                         idx_v[ibuf], isem)

    def wait_idx():
        pltpu.make_async_copy(idx_hbm.at[pl.ds(0, IDXR)], idx_v[0],
                              isem).wait()

    def reduce_chunk(b, pb):
        @pl.loop(0, OROWS)
        def _(orow):
            row0 = orow * s_len

            @pl.loop(0, 8)
            def _(g):
                goff = g * 16
                a = rows_v[b][row0, pl.ds(goff, 16)]
                bacc = rows_v[b][row0 + 1, pl.ds(goff, 16)]
                for s in range(2, s_len, 2):
                    a = a + rows_v[b][row0 + s, pl.ds(goff, 16)]
                    bacc = bacc + rows_v[b][row0 + s + 1, pl.ds(goff, 16)]
                pooled_v[pb][orow, pl.ds(goff, 16)] = a + bacc

    load_idx_sync(0, 0)
    fire_chunk(0, 0, 0)
    if n_super > 1:
        load_idx_async(1, 1)

    @pl.loop(0, n_super // 2)
    def _(gp):
        for ib in (0, 1):
            sb = gp * 2 + ib
            for c in range(8):
                b = c & 1
                ci = sb * 8 + c
                wait_gathers(b)
                if c == 0:
                    @pl.when(jnp.logical_and(sb >= 1, sb <= n_super - 2))
                    def _pf():
                        load_idx_async(sb + 1, 1 - ib)
                if c < 7:
                    fire_chunk(ib, c + 1, 1 - b)
                else:
                    @pl.when(sb <= n_super - 2)
                    def _nx():
                        wait_idx()
                        fire_chunk(1 - ib, 0, 1 - b)
                pb = c & 1
                if c >= 2:
                    wait_out(pb)
                else:
                    @pl.when(sb >= 1)
                    def _wo():
                        wait_out(pb)
                reduce_chunk(b, pb)
                pltpu.async_copy(pooled_v[pb],
                                 out_hbm.at[pl.ds(out_base + ci * OROWS,
                                                  OROWS)],
                                 osem[pb])
    wait_out(0)
    wait_out(1)


def _sc_gather_both(tri_table, tri_idx, int_table, int_idx, s_len):
    n_tri, n_int = tri_idx.shape[0], int_idx.shape[0]
    tri_pw, int_pw = n_tri // NW, n_int // NW
    tri_ns = tri_pw // 40
    int_ns = int_pw // SBR
    assert tri_pw % 40 == 0 and int_pw % SBR == 0 and int_ns % 2 == 0
    n_pool = n_tri * R // s_len
    D = tri_table.shape[1]

    mesh = plsc.VectorSubcoreMesh(core_axis_name="c", subcore_axis_name="s")

    @functools.partial(
        pl.kernel,
        mesh=mesh,
        out_type=(jax.ShapeDtypeStruct((n_pool, D), jnp.float32),
                  jax.ShapeDtypeStruct((n_int * R, D), jnp.float32)),
        scratch_types=[
            pltpu.VMEM((40, R), jnp.int32),
            pltpu.VMEM((40, R), jnp.int32),
            pltpu.VMEM((320, 128), jnp.float32),
            pltpu.VMEM((320, 128), jnp.float32),
            pltpu.VMEM((16, 128), jnp.float32),
            pltpu.VMEM((16, 128), jnp.float32),
            pltpu.SemaphoreType.DMA,
            pltpu.SemaphoreType.DMA,
            pltpu.SemaphoreType.DMA,
            pltpu.SemaphoreType.DMA,
            pltpu.SemaphoreType.DMA,
        ],
    )
    def gather_kernel(tri_t_hbm, tri_i_hbm, int_t_hbm, int_i_hbm,
                      tri_o_hbm, int_o_hbm,
                      idx0, idx1, rows0, rows1, pool0, pool1,
                      g0, g1, o0, o1, isem):
        wid = lax.axis_index("s") * NC + lax.axis_index("c")
        _emit_pooled_loop(tri_t_hbm, tri_i_hbm, tri_o_hbm, wid,
                          (idx0, idx1), (rows0, rows1), (pool0, pool1),
                          (g0, g1), (o0, o1), isem, tri_ns, s_len)
        _emit_table_loop(int_t_hbm, int_i_hbm, int_o_hbm,
                         wid * int_pw, wid * int_pw * R, int_ns,
                         (idx0.at[pl.ds(0, SBR)], idx1.at[pl.ds(0, SBR)]),
                         (rows0.at[pl.ds(0, CH)], rows1.at[pl.ds(0, CH)]),
                         (g0, g1), (o0, o1))

    return gather_kernel(tri_table, tri_idx, int_table, int_idx)


def _tc_mlp(xp, gi, true_l, w1t, b1, w2at, w2bt, b2, w3t, b3):
    B, F = xp.shape
    L = true_l
    BB = 256

    def body(xp_ref, gi_ref, w1t_ref, b1_ref, w2at_ref, w2bt_ref, b2_ref,
             w3t_ref, b3_ref, o_ref):
        t = jnp.dot(xp_ref[...], w1t_ref[...],
                    preferred_element_type=jnp.float32,
                    precision=jax.lax.Precision.HIGHEST)
        t = jnp.maximum(t + b1_ref[...], 0.0)
        acc2 = gi_ref[:, 0, :]
        for s in range(1, L):
            acc2 = acc2 + gi_ref[:, s, :]
        y = acc2 * (1.0 / L)
        hp = jax.lax.Precision.HIGHEST
        h = (jnp.dot(y, w2at_ref[...], preferred_element_type=jnp.float32,
                     precision=hp)
             + jnp.dot(t, w2bt_ref[...], preferred_element_type=jnp.float32,
                       precision=hp))
        h = jnp.maximum(h + b2_ref[...], 0.0)
        o_ref[...] = (jnp.dot(h, w3t_ref[...], preferred_element_type=jnp.float32,
                              precision=hp)
                      + b3_ref[...])

    return pl.pallas_call(
        body,
        grid=(B // BB,),
        in_specs=[
            pl.BlockSpec((BB, F), lambda i: (i, 0)),
            pl.BlockSpec((BB, gi.shape[1], gi.shape[2]), lambda i: (i, 0, 0)),
            pl.BlockSpec(w1t.shape, lambda i: (0, 0)),
            pl.BlockSpec(b1.shape, lambda i: (0, 0)),
            pl.BlockSpec(w2at.shape, lambda i: (0, 0)),
            pl.BlockSpec(w2bt.shape, lambda i: (0, 0)),
            pl.BlockSpec(b2.shape, lambda i: (0, 0)),
            pl.BlockSpec(w3t.shape, lambda i: (0, 0)),
            pl.BlockSpec(b3.shape, lambda i: (0, 0)),
        ],
        out_specs=pl.BlockSpec((BB, w3t.shape[1]), lambda i: (i, 0)),
        out_shape=jax.ShapeDtypeStruct((B, w3t.shape[1]), jnp.float32),
    )(xp, gi, w1t, b1, w2at, w2bt, b2, w3t, b3)


def kernel(trigram_ids, interacted_rate, trigram_table, subreddit_table,
           W1, b1, W2, b2, W3, b3):
    B, S, TRI = trigram_ids.shape
    L = interacted_rate.shape[1]
    EMB = trigram_table.shape[1]

    LP = 64
    tri_idx = (trigram_ids.astype(jnp.int32)
               .transpose(0, 2, 1).reshape(-1, 64))
    int_pad = jnp.pad(interacted_rate.astype(jnp.int32),
                      ((0, 0), (0, LP - L)))
    int_idx = int_pad.reshape(-1, 64)

    g_pool, g_int = _sc_gather_both(trigram_table, tri_idx,
                                    subreddit_table, int_idx, S)

    xp = g_pool.reshape(B, TRI * EMB)
    gi = g_int.reshape(B, LP, EMB)

    return _tc_mlp(
        xp, gi, L,
        W1.T * (1.0 / S), b1.reshape(1, -1),
        W2[:, :EMB].T, W2[:, EMB:].T, b2.reshape(1, -1),
        W3.T, b3.reshape(1, -1),
    )


# speedup vs baseline: 1.0071x; 1.0071x over previous
import functools

import jax
import jax.numpy as jnp
from jax import lax
from jax.experimental import pallas as pl
from jax.experimental.pallas import tpu as pltpu
from jax.experimental.pallas import tpu_sc as plsc

NC = 2
NS = 16
NW = NC * NS


K = 4
R = 64
CH = K * R
SBR = 16


def _emit_table_loop(table_hbm, idx_hbm, out_hbm, idx_w_base, out_w_base,
                     n_super, idx_v, rows_v, gsem, osem):
    n_chunks = n_super * 4

    def wait_out(b):
        pltpu.make_async_copy(rows_v[b], out_hbm.at[pl.ds(0, CH)], osem[b]).wait()

    def wait_gathers(b):
        pltpu.make_async_copy(out_hbm.at[pl.ds(0, CH)], rows_v[b], gsem[b]).wait()

    @pl.loop(0, n_super // 2)
    def _(gp):
        for ib in (0, 1):
            sb = gp * 2 + ib
            pltpu.sync_copy(idx_hbm.at[pl.ds(idx_w_base + sb * SBR, SBR)],
                            idx_v[ib])
            for c in range(4):
                b = c & 1
                if c >= 2:
                    wait_out(b)
                else:
                    @pl.when(sb >= 1)
                    def _w():
                        wait_out(b)
                for j in range(K):
                    pltpu.async_copy(
                        table_hbm.at[idx_v[ib].at[c * K + j]],
                        rows_v[b].at[pl.ds(j * R, R)],
                        gsem[b])
                prev_out = out_w_base + (sb * 4 + c - 1) * CH

                def _drain(prev_out=prev_out, b=b):
                    wait_gathers(1 - b)
                    pltpu.async_copy(rows_v[1 - b],
                                     out_hbm.at[pl.ds(prev_out, CH)],
                                     osem[1 - b])
                if c >= 1:
                    _drain()
                else:
                    @pl.when(sb >= 1)
                    def _d():
                        _drain()
    wait_gathers(1)
    pltpu.async_copy(
        rows_v[1],
        out_hbm.at[pl.ds(out_w_base + (n_chunks - 1) * CH, CH)], osem[1])
    wait_out(0)
    wait_out(1)


def _emit_pooled_loop(table_hbm, idx_hbm, out_hbm, wid, idx_v, rows_v,
                      pooled_v, gsem, osem, isem, n_super, s_len):
    IDXR = 40
    CKI = 5
    CROWS = CKI * R
    OROWS = CROWS // s_len
    per_w_idx = n_super * IDXR
    idx_base = wid * per_w_idx
    out_base = wid * (per_w_idx * R // s_len)

    def fire_chunk(ibuf, c, b):
        for j in range(CKI):
            pltpu.async_copy(
                table_hbm.at[idx_v[ibuf].at[c * CKI + j]],
                rows_v[b].at[pl.ds(j * R, R)],
                gsem[b])

    def wait_gathers(b):
        pltpu.make_async_copy(table_hbm.at[pl.ds(0, CROWS)], rows_v[b],
                              gsem[b]).wait()

    def wait_out(pb):
        pltpu.make_async_copy(pooled_v[pb], out_hbm.at[pl.ds(0, OROWS)],
                              osem[pb]).wait()

    def load_idx_sync(sb, ibuf):
        pltpu.sync_copy(idx_hbm.at[pl.ds(idx_base + sb * IDXR, IDXR)],
                        idx_v[ibuf])

    def load_idx_async(sb, ibuf):
        pltpu.async_copy(idx_hbm.at[pl.ds(idx_base + sb * IDXR, IDXR)],
                         idx_v[ibuf], isem)

    def wait_idx():
        pltpu.make_async_copy(idx_hbm.at[pl.ds(0, IDXR)], idx_v[0],
                              isem).wait()

    def reduce_chunk(b, pb):
        @pl.loop(0, OROWS)
        def _(orow):
            row0 = orow * s_len

            @pl.loop(0, 8)
            def _(g):
                goff = g * 16
                a = rows_v[b][row0, pl.ds(goff, 16)]
                bacc = rows_v[b][row0 + 1, pl.ds(goff, 16)]
                for s in range(2, s_len, 2):
                    a = a + rows_v[b][row0 + s, pl.ds(goff, 16)]
                    bacc = bacc + rows_v[b][row0 + s + 1, pl.ds(goff, 16)]
                pooled_v[pb][orow, pl.ds(goff, 16)] = a + bacc

    load_idx_sync(0, 0)
    fire_chunk(0, 0, 0)
    if n_super > 1:
        load_idx_async(1, 1)

    @pl.loop(0, n_super // 2)
    def _(gp):
        for ib in (0, 1):
            sb = gp * 2 + ib
            for c in range(8):
                b = c & 1
                ci = sb * 8 + c
                wait_gathers(b)
                if c == 0:
                    @pl.when(jnp.logical_and(sb >= 1, sb <= n_super - 2))
                    def _pf():
                        load_idx_async(sb + 1, 1 - ib)
                if c < 7:
                    fire_chunk(ib, c + 1, 1 - b)
                else:
                    @pl.when(sb <= n_super - 2)
                    def _nx():
                        wait_idx()
                        fire_chunk(1 - ib, 0, 1 - b)
                pb = c & 1
                if c >= 2:
                    wait_out(pb)
                else:
                    @pl.when(sb >= 1)
                    def _wo():
                        wait_out(pb)
                reduce_chunk(b, pb)
                pltpu.async_copy(pooled_v[pb],
                                 out_hbm.at[pl.ds(out_base + ci * OROWS,
                                                  OROWS)],
                                 osem[pb])
    wait_out(0)
    wait_out(1)


def _sc_gather_both(tri_table, tri_idx, int_table, int_idx, s_len):
    n_tri, n_int = tri_idx.shape[0], int_idx.shape[0]
    tri_pw, int_pw = n_tri // NW, n_int // NW
    tri_ns = tri_pw // 40
    int_ns = int_pw // SBR
    assert tri_pw % 40 == 0 and int_pw % SBR == 0 and int_ns % 2 == 0
    n_pool = n_tri * R // s_len
    D = tri_table.shape[1]

    mesh = plsc.VectorSubcoreMesh(core_axis_name="c", subcore_axis_name="s")

    @functools.partial(
        pl.kernel,
        mesh=mesh,
        out_type=(jax.ShapeDtypeStruct((n_pool, D), jnp.float32),
                  jax.ShapeDtypeStruct((n_int * R, D), jnp.float32)),
        scratch_types=[
            pltpu.VMEM((40, R), jnp.int32),
            pltpu.VMEM((40, R), jnp.int32),
            pltpu.VMEM((320, 128), jnp.float32),
            pltpu.VMEM((320, 128), jnp.float32),
            pltpu.VMEM((16, 128), jnp.float32),
            pltpu.VMEM((16, 128), jnp.float32),
            pltpu.SemaphoreType.DMA,
            pltpu.SemaphoreType.DMA,
            pltpu.SemaphoreType.DMA,
            pltpu.SemaphoreType.DMA,
            pltpu.SemaphoreType.DMA,
        ],
    )
    def gather_kernel(tri_t_hbm, tri_i_hbm, int_t_hbm, int_i_hbm,
                      tri_o_hbm, int_o_hbm,
                      idx0, idx1, rows0, rows1, pool0, pool1,
                      g0, g1, o0, o1, isem):
        wid = lax.axis_index("s") * NC + lax.axis_index("c")
        with jax.named_scope("tri_pooled_gather"):
            _emit_pooled_loop(tri_t_hbm, tri_i_hbm, tri_o_hbm, wid,
                              (idx0, idx1), (rows0, rows1), (pool0, pool1),
                              (g0, g1), (o0, o1), isem, tri_ns, s_len)
        with jax.named_scope("int_gather"):
            _emit_table_loop(int_t_hbm, int_i_hbm, int_o_hbm,
                             wid * int_pw, wid * int_pw * R, int_ns,
                             (idx0.at[pl.ds(0, SBR)], idx1.at[pl.ds(0, SBR)]),
                             (rows0.at[pl.ds(0, CH)], rows1.at[pl.ds(0, CH)]),
                             (g0, g1), (o0, o1))

    return gather_kernel(tri_table, tri_idx, int_table, int_idx)


def _tc_mlp(xp, gi, true_l, w1t, b1, w2at, w2bt, b2, w3t, b3):
    B, F = xp.shape
    L = true_l
    BB = 256

    def body(xp_ref, gi_ref, w1t_ref, b1_ref, w2at_ref, w2bt_ref, b2_ref,
             w3t_ref, b3_ref, o_ref):
        t = jnp.dot(xp_ref[...], w1t_ref[...],
                    preferred_element_type=jnp.float32,
                    precision=jax.lax.Precision.HIGHEST)
        t = jnp.maximum(t + b1_ref[...], 0.0)
        acc2 = gi_ref[:, 0, :]
        for s in range(1, L):
            acc2 = acc2 + gi_ref[:, s, :]
        y = acc2 * (1.0 / L)
        hp = jax.lax.Precision.HIGHEST
        h = (jnp.dot(y, w2at_ref[...], preferred_element_type=jnp.float32,
                     precision=hp)
             + jnp.dot(t, w2bt_ref[...], preferred_element_type=jnp.float32,
                       precision=hp))
        h = jnp.maximum(h + b2_ref[...], 0.0)
        o_ref[...] = (jnp.dot(h, w3t_ref[...], preferred_element_type=jnp.float32,
                              precision=hp)
                      + b3_ref[...])

    return pl.pallas_call(
        body,
        grid=(B // BB,),
        in_specs=[
            pl.BlockSpec((BB, F), lambda i: (i, 0)),
            pl.BlockSpec((BB, gi.shape[1], gi.shape[2]), lambda i: (i, 0, 0)),
            pl.BlockSpec(w1t.shape, lambda i: (0, 0)),
            pl.BlockSpec(b1.shape, lambda i: (0, 0)),
            pl.BlockSpec(w2at.shape, lambda i: (0, 0)),
            pl.BlockSpec(w2bt.shape, lambda i: (0, 0)),
            pl.BlockSpec(b2.shape, lambda i: (0, 0)),
            pl.BlockSpec(w3t.shape, lambda i: (0, 0)),
            pl.BlockSpec(b3.shape, lambda i: (0, 0)),
        ],
        out_specs=pl.BlockSpec((BB, w3t.shape[1]), lambda i: (i, 0)),
        out_shape=jax.ShapeDtypeStruct((B, w3t.shape[1]), jnp.float32),
    )(xp, gi, w1t, b1, w2at, w2bt, b2, w3t, b3)


def kernel(trigram_ids, interacted_rate, trigram_table, subreddit_table,
           W1, b1, W2, b2, W3, b3):
    B, S, TRI = trigram_ids.shape
    L = interacted_rate.shape[1]
    EMB = trigram_table.shape[1]

    LP = 64
    tri_idx = (trigram_ids.astype(jnp.int32)
               .transpose(0, 2, 1).reshape(-1, 64))
    int_pad = jnp.pad(interacted_rate.astype(jnp.int32),
                      ((0, 0), (0, LP - L)))
    int_idx = int_pad.reshape(-1, 64)

    g_pool, g_int = _sc_gather_both(trigram_table, tri_idx,
                                    subreddit_table, int_idx, S)

    xp = g_pool.reshape(B, TRI * EMB)
    gi = g_int.reshape(B, LP, EMB)

    return _tc_mlp(
        xp, gi, L,
        W1.T * (1.0 / S), b1.reshape(1, -1),
        W2[:, :EMB].T, W2[:, EMB:].T, b2.reshape(1, -1),
        W3.T, b3.reshape(1, -1),
    )


# speedup vs baseline: 2.1735x; 2.1582x over previous
import functools

import jax
import jax.numpy as jnp
from jax import lax
from jax.experimental import pallas as pl
from jax.experimental.pallas import tpu as pltpu
from jax.experimental.pallas import tpu_sc as plsc

NC = 2
NS = 16
NW = NC * NS


K = 4
R = 64
CH = K * R
SBR = 16


def _emit_table_loop(table_hbm, idx_hbm, out_hbm, idx_w_base, out_w_base,
                     n_super, idx_v, rows_v, gsem, osem):
    n_chunks = n_super * 4

    def wait_out(b):
        pltpu.make_async_copy(rows_v[b], out_hbm.at[pl.ds(0, CH)], osem[b]).wait()

    def wait_gathers(b):
        pltpu.make_async_copy(out_hbm.at[pl.ds(0, CH)], rows_v[b], gsem[b]).wait()

    @pl.loop(0, n_super // 2)
    def _(gp):
        for ib in (0, 1):
            sb = gp * 2 + ib
            pltpu.sync_copy(idx_hbm.at[pl.ds(idx_w_base + sb * SBR, SBR)],
                            idx_v[ib])
            for c in range(4):
                b = c & 1
                if c >= 2:
                    wait_out(b)
                else:
                    @pl.when(sb >= 1)
                    def _w():
                        wait_out(b)
                for j in range(K):
                    pltpu.async_copy(
                        table_hbm.at[idx_v[ib].at[c * K + j]],
                        rows_v[b].at[pl.ds(j * R, R)],
                        gsem[b])
                prev_out = out_w_base + (sb * 4 + c - 1) * CH

                def _drain(prev_out=prev_out, b=b):
                    wait_gathers(1 - b)
                    pltpu.async_copy(rows_v[1 - b],
                                     out_hbm.at[pl.ds(prev_out, CH)],
                                     osem[1 - b])
                if c >= 1:
                    _drain()
                else:
                    @pl.when(sb >= 1)
                    def _d():
                        _drain()
    wait_gathers(1)
    pltpu.async_copy(
        rows_v[1],
        out_hbm.at[pl.ds(out_w_base + (n_chunks - 1) * CH, CH)], osem[1])
    wait_out(0)
    wait_out(1)


def _emit_pooled_loop(table_hbm, idx_hbm, out_hbm, wid, idx_v, rows_v,
                      pooled_v, gsem, osem, isem, n_super, s_len):
    IDXR = 40
    CKI = 5
    CROWS = CKI * R
    OROWS = CROWS // s_len
    per_w_idx = n_super * IDXR
    idx_base = wid * per_w_idx
    out_base = wid * (per_w_idx * R // s_len)

    def fire_chunk(ibuf, c, b):
        for j in range(CKI):
            pltpu.async_copy(
                table_hbm.at[idx_v[ibuf].at[c * CKI + j]],
                rows_v[b].at[pl.ds(j * R, R)],
                gsem[b])

    def wait_gathers(b):
        pltpu.make_async_copy(table_hbm.at[pl.ds(0, CROWS)], rows_v[b],
                              gsem[b]).wait()

    def wait_out(pb):
        pltpu.make_async_copy(pooled_v[pb], out_hbm.at[pl.ds(0, OROWS)],
                              osem[pb]).wait()

    def load_idx_sync(sb, ibuf):
        pltpu.sync_copy(idx_hbm.at[pl.ds(idx_base + sb * IDXR, IDXR)],
                        idx_v[ibuf])

    def load_idx_async(sb, ibuf):
        pltpu.async_copy(idx_hbm.at[pl.ds(idx_base + sb * IDXR, IDXR)],
                         idx_v[ibuf], isem)

    def wait_idx():
        pltpu.make_async_copy(idx_hbm.at[pl.ds(0, IDXR)], idx_v[0],
                              isem).wait()

    def reduce_chunk(b, pb):
        @pl.loop(0, OROWS)
        def _(orow):
            row0 = orow * s_len

            @pl.loop(0, 8)
            def _(g):
                goff = g * 16
                a = rows_v[b][row0, pl.ds(goff, 16)]
                bacc = rows_v[b][row0 + 1, pl.ds(goff, 16)]
                for s in range(2, s_len, 2):
                    a = a + rows_v[b][row0 + s, pl.ds(goff, 16)]
                    bacc = bacc + rows_v[b][row0 + s + 1, pl.ds(goff, 16)]
                pooled_v[pb][orow, pl.ds(goff, 16)] = a + bacc

    load_idx_sync(0, 0)
    fire_chunk(0, 0, 0)
    if n_super > 1:
        load_idx_async(1, 1)

    @pl.loop(0, n_super // 2)
    def _(gp):
        for ib in (0, 1):
            sb = gp * 2 + ib
            for c in range(8):
                b = c & 1
                ci = sb * 8 + c
                wait_gathers(b)
                if c == 0:
                    @pl.when(jnp.logical_and(sb >= 1, sb <= n_super - 2))
                    def _pf():
                        load_idx_async(sb + 1, 1 - ib)
                if c < 7:
                    fire_chunk(ib, c + 1, 1 - b)
                else:
                    @pl.when(sb <= n_super - 2)
                    def _nx():
                        wait_idx()
                        fire_chunk(1 - ib, 0, 1 - b)
                pb = c & 1
                if c >= 2:
                    wait_out(pb)
                else:
                    @pl.when(sb >= 1)
                    def _wo():
                        wait_out(pb)
                reduce_chunk(b, pb)
                pltpu.async_copy(pooled_v[pb],
                                 out_hbm.at[pl.ds(out_base + ci * OROWS,
                                                  OROWS)],
                                 osem[pb])
    wait_out(0)
    wait_out(1)


def _sc_gather_both(tri_table, tri_idx, int_table, int_idx, s_len):
    n_tri, n_int = tri_idx.shape[0], int_idx.shape[0]
    tri_pw, int_pw = n_tri // NW, n_int // NW
    tri_ns = tri_pw // 40
    int_ns = int_pw // SBR
    assert tri_pw % 40 == 0 and int_pw % SBR == 0 and int_ns % 2 == 0
    n_pool = n_tri * R // s_len
    D = tri_table.shape[1]

    mesh = plsc.VectorSubcoreMesh(core_axis_name="c", subcore_axis_name="s")

    @functools.partial(
        pl.kernel,
        mesh=mesh,
        out_type=(jax.ShapeDtypeStruct((n_pool, D), jnp.float32),
                  jax.ShapeDtypeStruct((n_int * R, D), jnp.float32)),
        scratch_types=[
            pltpu.VMEM((40, R), jnp.int32),
            pltpu.VMEM((40, R), jnp.int32),
            pltpu.VMEM((320, 128), jnp.float32),
            pltpu.VMEM((320, 128), jnp.float32),
            pltpu.VMEM((16, 128), jnp.float32),
            pltpu.VMEM((16, 128), jnp.float32),
            pltpu.SemaphoreType.DMA,
            pltpu.SemaphoreType.DMA,
            pltpu.SemaphoreType.DMA,
            pltpu.SemaphoreType.DMA,
            pltpu.SemaphoreType.DMA,
        ],
    )
    def gather_kernel(tri_t_hbm, tri_i_hbm, int_t_hbm, int_i_hbm,
                      tri_o_hbm, int_o_hbm,
                      idx0, idx1, rows0, rows1, pool0, pool1,
                      g0, g1, o0, o1, isem):
        wid = lax.axis_index("s") * NC + lax.axis_index("c")
        with jax.named_scope("tri_pooled_gather"):
            _emit_pooled_loop(tri_t_hbm, tri_i_hbm, tri_o_hbm, wid,
                              (idx0, idx1), (rows0, rows1), (pool0, pool1),
                              (g0, g1), (o0, o1), isem, tri_ns, s_len)
        with jax.named_scope("int_gather"):
            _emit_table_loop(int_t_hbm, int_i_hbm, int_o_hbm,
                             wid * int_pw, wid * int_pw * R, int_ns,
                             (idx0.at[pl.ds(0, SBR)], idx1.at[pl.ds(0, SBR)]),
                             (rows0.at[pl.ds(0, CH)], rows1.at[pl.ds(0, CH)]),
                             (g0, g1), (o0, o1))

    return gather_kernel(tri_table, tri_idx, int_table, int_idx)


def _tc_mlp(xp, gi, true_l, w1t, b1, w2at, w2bt, b2, w3t, b3):
    B, F = xp.shape
    L = true_l
    BB = 256

    def body(xp_ref, gi_ref, w1t_ref, b1_ref, w2at_ref, w2bt_ref, b2_ref,
             w3t_ref, b3_ref, o_ref):
        t = jnp.dot(xp_ref[...], w1t_ref[...],
                    preferred_element_type=jnp.float32,
                    precision=jax.lax.Precision.HIGHEST)
        t = jnp.maximum(t + b1_ref[...], 0.0)
        acc2 = gi_ref[:, 0, :]
        for s in range(1, L):
            acc2 = acc2 + gi_ref[:, s, :]
        y = acc2 * (1.0 / L)
        hp = jax.lax.Precision.HIGHEST
        h = (jnp.dot(y, w2at_ref[...], preferred_element_type=jnp.float32,
                     precision=hp)
             + jnp.dot(t, w2bt_ref[...], preferred_element_type=jnp.float32,
                       precision=hp))
        h = jnp.maximum(h + b2_ref[...], 0.0)
        o_ref[...] = (jnp.dot(h, w3t_ref[...], preferred_element_type=jnp.float32,
                              precision=hp)
                      + b3_ref[...])

    return pl.pallas_call(
        body,
        grid=(B // BB,),
        in_specs=[
            pl.BlockSpec((BB, F), lambda i: (i, 0)),
            pl.BlockSpec((BB, gi.shape[1], gi.shape[2]), lambda i: (i, 0, 0)),
            pl.BlockSpec(w1t.shape, lambda i: (0, 0)),
            pl.BlockSpec(b1.shape, lambda i: (0, 0)),
            pl.BlockSpec(w2at.shape, lambda i: (0, 0)),
            pl.BlockSpec(w2bt.shape, lambda i: (0, 0)),
            pl.BlockSpec(b2.shape, lambda i: (0, 0)),
            pl.BlockSpec(w3t.shape, lambda i: (0, 0)),
            pl.BlockSpec(b3.shape, lambda i: (0, 0)),
        ],
        out_specs=pl.BlockSpec((BB, w3t.shape[1]), lambda i: (i, 0)),
        out_shape=jax.ShapeDtypeStruct((B, w3t.shape[1]), jnp.float32),
    )(xp, gi, w1t, b1, w2at, w2bt, b2, w3t, b3)


def kernel(trigram_ids, interacted_rate, trigram_table, subreddit_table,
           W1, b1, W2, b2, W3, b3):
    B, S, TRI = trigram_ids.shape
    L = interacted_rate.shape[1]
    EMB = trigram_table.shape[1]

    LP = 64
    tri_idx = (trigram_ids.astype(jnp.int32)
               .transpose(0, 2, 1).reshape(-1, 64))
    ir32 = interacted_rate.astype(jnp.int32)
    int_pad = jnp.concatenate([ir32, ir32[:, :LP - L]], axis=1)
    int_idx = int_pad.reshape(-1, 64)

    g_pool, g_int = _sc_gather_both(trigram_table, tri_idx,
                                    subreddit_table, int_idx, S)

    xp = g_pool.reshape(B, TRI * EMB)
    gi = g_int.reshape(B, LP, EMB)

    return _tc_mlp(
        xp, gi, L,
        W1.T * (1.0 / S), b1.reshape(1, -1),
        W2[:, :EMB].T, W2[:, EMB:].T, b2.reshape(1, -1),
        W3.T, b3.reshape(1, -1),
    )


# speedup vs baseline: 2.2797x; 1.0489x over previous
import dataclasses
import functools

import jax
import jax.numpy as jnp
from jax import lax
from jax.experimental import pallas as pl
from jax.experimental.pallas import tpu as pltpu
from jax.experimental.pallas import tpu_sc as plsc

NC = 2
NS = 16
NW = NC * NS


K = 4
R = 64
CH = K * R
SBR = 16


def _emit_table_loop(table_hbm, idx_hbm, out_hbm, idx_w_base, out_w_base,
                     n_super, idx_v, rows_v, gsem, osem):
    n_chunks = n_super * 4

    def wait_out(b):
        pltpu.make_async_copy(rows_v[b], out_hbm.at[pl.ds(0, CH)], osem[b]).wait()

    def wait_gathers(b):
        pltpu.make_async_copy(out_hbm.at[pl.ds(0, CH)], rows_v[b], gsem[b]).wait()

    @pl.loop(0, n_super // 2)
    def _(gp):
        for ib in (0, 1):
            sb = gp * 2 + ib
            pltpu.sync_copy(idx_hbm.at[pl.ds(idx_w_base + sb * SBR, SBR)],
                            idx_v[ib])
            for c in range(4):
                b = c & 1
                if c >= 2:
                    wait_out(b)
                else:
                    @pl.when(sb >= 1)
                    def _w():
                        wait_out(b)
                for j in range(K):
                    pltpu.async_copy(
                        table_hbm.at[idx_v[ib].at[c * K + j]],
                        rows_v[b].at[pl.ds(j * R, R)],
                        gsem[b])
                prev_out = out_w_base + (sb * 4 + c - 1) * CH

                def _drain(prev_out=prev_out, b=b):
                    wait_gathers(1 - b)
                    pltpu.async_copy(rows_v[1 - b],
                                     out_hbm.at[pl.ds(prev_out, CH)],
                                     osem[1 - b])
                if c >= 1:
                    _drain()
                else:
                    @pl.when(sb >= 1)
                    def _d():
                        _drain()
    wait_gathers(1)
    pltpu.async_copy(
        rows_v[1],
        out_hbm.at[pl.ds(out_w_base + (n_chunks - 1) * CH, CH)], osem[1])
    wait_out(0)
    wait_out(1)


def _emit_pooled_loop(table_hbm, idx_hbm, out_hbm, wid, idx_v, idx_t, rows_v,
                      pooled_v, gsem, osem, isem, n_super, s_len):
    IDXR = 40
    CKI = 5
    CROWS = CKI * R
    OROWS = CROWS // s_len
    per_w_idx = n_super * IDXR
    idx_base = wid * per_w_idx
    out_base = wid * (per_w_idx * R // s_len)
    TPB = s_len * R

    def fire_chunk(ibuf, c, b):
        for j in range(CKI):
            pltpu.async_copy(
                table_hbm.at[idx_t[ibuf].at[pl.ds((c * CKI + j) * R, R)]],
                rows_v[b].at[pl.ds(j * R, R)],
                gsem[b])

    def wait_gathers(b):
        pltpu.make_async_copy(table_hbm.at[pl.ds(0, CROWS)], rows_v[b],
                              gsem[b]).wait()

    def wait_out(pb):
        pltpu.make_async_copy(pooled_v[pb], out_hbm.at[pl.ds(0, OROWS)],
                              osem[pb]).wait()

    def load_idx_sync(sb, ibuf):
        pltpu.sync_copy(idx_hbm.at[pl.ds(idx_base + sb * IDXR, IDXR)],
                        idx_v[ibuf])

    def load_idx_async(sb, ibuf):
        pltpu.async_copy(idx_hbm.at[pl.ds(idx_base + sb * IDXR, IDXR)],
                         idx_v[ibuf], isem)

    def wait_idx():
        pltpu.make_async_copy(idx_hbm.at[pl.ds(0, IDXR)], idx_v[0],
                              isem).wait()

    lane = jax.lax.iota(jnp.int32, 16)

    def transpose_idx(ibuf):
        for bi in (0, 1):
            for g in range(4):

                @pl.loop(0, s_len)
                def _(s):
                    v = idx_v[ibuf][bi * s_len + s, pl.ds(g * 16, 16)]
                    dst = (bi * TPB + g * 16 * s_len + s) + lane * s_len
                    plsc.store_scatter(idx_t[ibuf], [dst], v)

    def reduce_chunk(b, pb):
        @pl.loop(0, OROWS)
        def _(orow):
            row0 = orow * s_len

            @pl.loop(0, 8)
            def _(g):
                goff = g * 16
                a = rows_v[b][row0, pl.ds(goff, 16)]
                bacc = rows_v[b][row0 + 1, pl.ds(goff, 16)]
                for s in range(2, s_len, 2):
                    a = a + rows_v[b][row0 + s, pl.ds(goff, 16)]
                    bacc = bacc + rows_v[b][row0 + s + 1, pl.ds(goff, 16)]
                pooled_v[pb][orow, pl.ds(goff, 16)] = a + bacc

    load_idx_sync(0, 0)
    transpose_idx(0)
    fire_chunk(0, 0, 0)
    if n_super > 1:
        load_idx_async(1, 1)

    @pl.loop(0, n_super // 2)
    def _(gp):
        for ib in (0, 1):
            sb = gp * 2 + ib
            for c in range(8):
                b = c & 1
                ci = sb * 8 + c
                wait_gathers(b)
                if c == 0:
                    @pl.when(jnp.logical_and(sb >= 1, sb <= n_super - 2))
                    def _pf():
                        load_idx_async(sb + 1, 1 - ib)
                if c < 7:
                    fire_chunk(ib, c + 1, 1 - b)
                else:
                    @pl.when(sb <= n_super - 2)
                    def _nx():
                        wait_idx()
                        transpose_idx(1 - ib)
                        fire_chunk(1 - ib, 0, 1 - b)
                pb = c & 1
                if c >= 2:
                    wait_out(pb)
                else:
                    @pl.when(sb >= 1)
                    def _wo():
                        wait_out(pb)
                reduce_chunk(b, pb)
                pltpu.async_copy(pooled_v[pb],
                                 out_hbm.at[pl.ds(out_base + ci * OROWS,
                                                  OROWS)],
                                 osem[pb])
    wait_out(0)
    wait_out(1)


def _sc_gather_both(tri_table, tri_idx, int_table, int_idx, s_len):
    n_tri, n_int = tri_idx.shape[0], int_idx.shape[0]
    tri_pw, int_pw = n_tri // NW, n_int // NW
    tri_ns = tri_pw // 40
    int_ns = int_pw // SBR
    assert tri_pw % 40 == 0 and int_pw % SBR == 0 and int_ns % 2 == 0
    n_pool = n_tri * R // s_len
    D = tri_table.shape[1]

    mesh = plsc.VectorSubcoreMesh(core_axis_name="c", subcore_axis_name="s")
    cp = pltpu.CompilerParams()
    if "needs_layout_passes" in pltpu.CompilerParams.__dataclass_fields__:
        cp = dataclasses.replace(cp, needs_layout_passes=False)

    @functools.partial(
        pl.kernel,
        mesh=mesh,
        compiler_params=cp,
        out_type=(jax.ShapeDtypeStruct((n_pool, D), jnp.float32),
                  jax.ShapeDtypeStruct((n_int * R, D), jnp.float32)),
        scratch_types=[
            pltpu.VMEM((40, R), jnp.int32),
            pltpu.VMEM((40, R), jnp.int32),
            pltpu.VMEM((2560,), jnp.int32),
            pltpu.VMEM((2560,), jnp.int32),
            pltpu.VMEM((320, 128), jnp.float32),
            pltpu.VMEM((320, 128), jnp.float32),
            pltpu.VMEM((16, 128), jnp.float32),
            pltpu.VMEM((16, 128), jnp.float32),
            pltpu.SemaphoreType.DMA,
            pltpu.SemaphoreType.DMA,
            pltpu.SemaphoreType.DMA,
            pltpu.SemaphoreType.DMA,
            pltpu.SemaphoreType.DMA,
        ],
    )
    def gather_kernel(tri_t_hbm, tri_i_hbm, int_t_hbm, int_i_hbm,
                      tri_o_hbm, int_o_hbm,
                      idx0, idx1, idxt0, idxt1, rows0, rows1, pool0, pool1,
                      g0, g1, o0, o1, isem):
        wid = lax.axis_index("s") * NC + lax.axis_index("c")
        with jax.named_scope("tri_pooled_gather"):
            _emit_pooled_loop(tri_t_hbm, tri_i_hbm, tri_o_hbm, wid,
                              (idx0, idx1), (idxt0, idxt1),
                              (rows0, rows1), (pool0, pool1),
                              (g0, g1), (o0, o1), isem, tri_ns, s_len)
        with jax.named_scope("int_gather"):
            _emit_table_loop(int_t_hbm, int_i_hbm, int_o_hbm,
                             wid * int_pw, wid * int_pw * R, int_ns,
                             (idx0.at[pl.ds(0, SBR)], idx1.at[pl.ds(0, SBR)]),
                             (rows0.at[pl.ds(0, CH)], rows1.at[pl.ds(0, CH)]),
                             (g0, g1), (o0, o1))

    return gather_kernel(tri_table, tri_idx, int_table, int_idx)


def _tc_mlp(xp, gi, true_l, w1t, b1, w2at, w2bt, b2, w3t, b3):
    B, F = xp.shape
    L = true_l
    BB = 256
    hp = jax.lax.Precision.HIGHEST

    def body(xp_ref, gi_ref, w1t_ref, b1_ref, w2at_ref, w2bt_ref, b2_ref,
             w3t_ref, b3_ref, o_ref):
        t = jnp.dot(xp_ref[...], w1t_ref[...],
                    preferred_element_type=jnp.float32, precision=hp)
        t = jnp.maximum(t + b1_ref[...], 0.0)
        acc2 = gi_ref[:, 0, :]
        for s in range(1, L):
            acc2 = acc2 + gi_ref[:, s, :]
        y = acc2 * (1.0 / L)
        h = (jnp.dot(y, w2at_ref[...], preferred_element_type=jnp.float32,
                     precision=hp)
             + jnp.dot(t, w2bt_ref[...], preferred_element_type=jnp.float32,
                       precision=hp))
        h = jnp.maximum(h + b2_ref[...], 0.0)
        o_ref[...] = (jnp.dot(h, w3t_ref[...], preferred_element_type=jnp.float32,
                              precision=hp)
                      + b3_ref[...])

    return pl.pallas_call(
        body,
        grid=(B // BB,),
        in_specs=[
            pl.BlockSpec((BB, F), lambda i: (i, 0)),
            pl.BlockSpec((BB, gi.shape[1], gi.shape[2]), lambda i: (i, 0, 0)),
            pl.BlockSpec(w1t.shape, lambda i: (0, 0)),
            pl.BlockSpec(b1.shape, lambda i: (0, 0)),
            pl.BlockSpec(w2at.shape, lambda i: (0, 0)),
            pl.BlockSpec(w2bt.shape, lambda i: (0, 0)),
            pl.BlockSpec(b2.shape, lambda i: (0, 0)),
            pl.BlockSpec(w3t.shape, lambda i: (0, 0)),
            pl.BlockSpec(b3.shape, lambda i: (0, 0)),
        ],
        out_specs=pl.BlockSpec((BB, w3t.shape[1]), lambda i: (i, 0)),
        out_shape=jax.ShapeDtypeStruct((B, w3t.shape[1]), jnp.float32),
    )(xp, gi, w1t, b1, w2at, w2bt, b2, w3t, b3)


def kernel(trigram_ids, interacted_rate, trigram_table, subreddit_table,
           W1, b1, W2, b2, W3, b3):
    B, S, TRI = trigram_ids.shape
    L = interacted_rate.shape[1]
    EMB = trigram_table.shape[1]

    LP = 64
    tri_idx = trigram_ids.astype(jnp.int32).reshape(-1, 64)
    ir32 = interacted_rate.astype(jnp.int32)
    int_pad = jnp.concatenate([ir32, ir32[:, :LP - L]], axis=1)
    int_idx = int_pad.reshape(-1, 64)

    g_pool, g_int = _sc_gather_both(trigram_table, tri_idx,
                                    subreddit_table, int_idx, S)

    xp = g_pool.reshape(B, TRI * EMB)
    gi = g_int.reshape(B, LP, EMB)

    return _tc_mlp(
        xp, gi, L,
        W1.T * (1.0 / S), b1.reshape(1, -1),
        W2[:, :EMB].T, W2[:, EMB:].T, b2.reshape(1, -1),
        W3.T, b3.reshape(1, -1),
    )


# speedup vs baseline: 2.3437x; 1.0281x over previous
import dataclasses
import functools

import jax
import jax.numpy as jnp
from jax import lax
from jax.experimental import pallas as pl
from jax.experimental.pallas import tpu as pltpu
from jax.experimental.pallas import tpu_sc as plsc

NC = 2
NS = 16
NW = NC * NS


K = 4
R = 64
CH = K * R
SBR = 16


def _emit_table_loop(table_hbm, idx_hbm, out_hbm, idx_w_base, out_w_base,
                     n_super, idx_v, rows_v, gsem, osem):
    n_chunks = n_super * 4

    def wait_out(b):
        pltpu.make_async_copy(rows_v[b], out_hbm.at[pl.ds(0, CH)], osem[b]).wait()

    def wait_gathers(b):
        pltpu.make_async_copy(out_hbm.at[pl.ds(0, CH)], rows_v[b], gsem[b]).wait()

    @pl.loop(0, n_super // 2)
    def _(gp):
        for ib in (0, 1):
            sb = gp * 2 + ib
            pltpu.sync_copy(idx_hbm.at[pl.ds(idx_w_base + sb * SBR, SBR)],
                            idx_v[ib])
            for c in range(4):
                b = c & 1
                if c >= 2:
                    wait_out(b)
                else:
                    @pl.when(sb >= 1)
                    def _w():
                        wait_out(b)
                for j in range(K):
                    pltpu.async_copy(
                        table_hbm.at[idx_v[ib].at[c * K + j]],
                        rows_v[b].at[pl.ds(j * R, R)],
                        gsem[b])
                prev_out = out_w_base + (sb * 4 + c - 1) * CH

                def _drain(prev_out=prev_out, b=b):
                    wait_gathers(1 - b)
                    pltpu.async_copy(rows_v[1 - b],
                                     out_hbm.at[pl.ds(prev_out, CH)],
                                     osem[1 - b])
                if c >= 1:
                    _drain()
                else:
                    @pl.when(sb >= 1)
                    def _d():
                        _drain()
    wait_gathers(1)
    pltpu.async_copy(
        rows_v[1],
        out_hbm.at[pl.ds(out_w_base + (n_chunks - 1) * CH, CH)], osem[1])
    wait_out(0)
    wait_out(1)


def _emit_pooled_loop(table_hbm, idx_hbm, out_hbm, wid, idx_v, idx_t, rows_v,
                      pooled_v, gsem, osem, isem, n_super, s_len):
    CKI = 5
    CROWS = CKI * R
    OROWS = CROWS // s_len
    out_base = wid * (n_super * 2 * R)
    TPB = s_len * R

    def fire_chunk(ibuf, c, b):
        for j in range(CKI):
            pltpu.async_copy(
                table_hbm.at[idx_t[ibuf].at[pl.ds((c * CKI + j) * R, R)]],
                rows_v[b].at[pl.ds(j * R, R)],
                gsem[b])

    def wait_gathers(b):
        pltpu.make_async_copy(table_hbm.at[pl.ds(0, CROWS)], rows_v[b],
                              gsem[b]).wait()

    def wait_out(pb):
        pltpu.make_async_copy(pooled_v[pb], out_hbm.at[pl.ds(0, OROWS)],
                              osem[pb]).wait()

    def load_idx_sync(sb, ibuf):
        b0 = wid * 2 * n_super + sb * 2
        pltpu.sync_copy(idx_hbm.at[b0], idx_v[ibuf].at[pl.ds(0, s_len)])
        pltpu.sync_copy(idx_hbm.at[b0 + 1],
                        idx_v[ibuf].at[pl.ds(s_len, s_len)])

    def load_idx_async(sb, ibuf):
        b0 = wid * 2 * n_super + sb * 2
        pltpu.async_copy(idx_hbm.at[b0], idx_v[ibuf].at[pl.ds(0, s_len)],
                         isem)
        pltpu.async_copy(idx_hbm.at[b0 + 1],
                         idx_v[ibuf].at[pl.ds(s_len, s_len)], isem)

    def wait_idx():
        for _ in range(2):
            pltpu.make_async_copy(idx_hbm.at[0],
                                  idx_v[0].at[pl.ds(0, s_len)], isem).wait()

    lane = jax.lax.iota(jnp.int32, 16)

    def transpose_idx(ibuf):
        for bi in (0, 1):
            for g in range(4):

                @pl.loop(0, s_len)
                def _(s):
                    v = idx_v[ibuf][bi * s_len + s, pl.ds(g * 16, 16)]
                    dst = (bi * TPB + g * 16 * s_len + s) + lane * s_len
                    plsc.store_scatter(idx_t[ibuf], [dst], v)

    def reduce_chunk(b, pb):
        @pl.loop(0, OROWS)
        def _(orow):
            row0 = orow * s_len

            @pl.loop(0, 8)
            def _(g):
                goff = g * 16
                a = rows_v[b][row0, pl.ds(goff, 16)]
                bacc = rows_v[b][row0 + 1, pl.ds(goff, 16)]
                for s in range(2, s_len, 2):
                    a = a + rows_v[b][row0 + s, pl.ds(goff, 16)]
                    bacc = bacc + rows_v[b][row0 + s + 1, pl.ds(goff, 16)]
                pooled_v[pb][orow, pl.ds(goff, 16)] = a + bacc

    load_idx_sync(0, 0)
    transpose_idx(0)
    fire_chunk(0, 0, 0)
    if n_super > 1:
        load_idx_async(1, 1)

    @pl.loop(0, n_super // 2)
    def _(gp):
        for ib in (0, 1):
            sb = gp * 2 + ib
            for c in range(8):
                b = c & 1
                ci = sb * 8 + c
                wait_gathers(b)
                if c == 0:
                    @pl.when(jnp.logical_and(sb >= 1, sb <= n_super - 2))
                    def _pf():
                        load_idx_async(sb + 1, 1 - ib)
                if c < 7:
                    fire_chunk(ib, c + 1, 1 - b)
                else:
                    @pl.when(sb <= n_super - 2)
                    def _nx():
                        wait_idx()
                        transpose_idx(1 - ib)
                        fire_chunk(1 - ib, 0, 1 - b)
                pb = c & 1
                if c >= 2:
                    wait_out(pb)
                else:
                    @pl.when(sb >= 1)
                    def _wo():
                        wait_out(pb)
                reduce_chunk(b, pb)
                pltpu.async_copy(pooled_v[pb],
                                 out_hbm.at[pl.ds(out_base + ci * OROWS,
                                                  OROWS)],
                                 osem[pb])
    wait_out(0)
    wait_out(1)


def _sc_gather_both(tri_table, tri_idx, int_table, int_idx, s_len):
    BT, _, TRI_ = tri_idx.shape
    n_int = int_idx.shape[0]
    int_pw = n_int // NW
    tri_ns = BT // NW // 2
    int_ns = int_pw // SBR
    assert BT % (NW * 2) == 0 and int_pw % SBR == 0 and int_ns % 2 == 0
    n_pool = BT * TRI_
    D = tri_table.shape[1]

    mesh = plsc.VectorSubcoreMesh(core_axis_name="c", subcore_axis_name="s")
    cp = pltpu.CompilerParams()
    if "needs_layout_passes" in pltpu.CompilerParams.__dataclass_fields__:
        cp = dataclasses.replace(cp, needs_layout_passes=False)

    @functools.partial(
        pl.kernel,
        mesh=mesh,
        compiler_params=cp,
        out_type=(jax.ShapeDtypeStruct((n_pool, D), jnp.float32),
                  jax.ShapeDtypeStruct((n_int * R, D), jnp.float32)),
        scratch_types=[
            pltpu.VMEM((2 * s_len, R), jnp.int32),
            pltpu.VMEM((2 * s_len, R), jnp.int32),
            pltpu.VMEM((SBR, R), jnp.int32),
            pltpu.VMEM((SBR, R), jnp.int32),
            pltpu.VMEM((2560,), jnp.int32),
            pltpu.VMEM((2560,), jnp.int32),
            pltpu.VMEM((320, 128), jnp.float32),
            pltpu.VMEM((320, 128), jnp.float32),
            pltpu.VMEM((16, 128), jnp.float32),
            pltpu.VMEM((16, 128), jnp.float32),
            pltpu.SemaphoreType.DMA,
            pltpu.SemaphoreType.DMA,
            pltpu.SemaphoreType.DMA,
            pltpu.SemaphoreType.DMA,
            pltpu.SemaphoreType.DMA,
        ],
    )
    def gather_kernel(tri_t_hbm, tri_i_hbm, int_t_hbm, int_i_hbm,
                      tri_o_hbm, int_o_hbm,
                      idx0, idx1, iidx0, iidx1, idxt0, idxt1,
                      rows0, rows1, pool0, pool1,
                      g0, g1, o0, o1, isem):
        wid = lax.axis_index("s") * NC + lax.axis_index("c")
        with jax.named_scope("tri_pooled_gather"):
            _emit_pooled_loop(tri_t_hbm, tri_i_hbm, tri_o_hbm, wid,
                              (idx0, idx1), (idxt0, idxt1),
                              (rows0, rows1), (pool0, pool1),
                              (g0, g1), (o0, o1), isem, tri_ns, s_len)
        with jax.named_scope("int_gather"):
            _emit_table_loop(int_t_hbm, int_i_hbm, int_o_hbm,
                             wid * int_pw, wid * int_pw * R, int_ns,
                             (iidx0, iidx1),
                             (rows0.at[pl.ds(0, CH)], rows1.at[pl.ds(0, CH)]),
                             (g0, g1), (o0, o1))

    return gather_kernel(tri_table, tri_idx, int_table, int_idx)


def _tc_mlp(xp, gi, true_l, w1t, b1, w2at, w2bt, b2, w3t, b3):
    B, F = xp.shape
    L = true_l
    BB = 256
    hp = jax.lax.Precision.HIGHEST

    def body(xp_ref, gi_ref, w1t_ref, b1_ref, w2at_ref, w2bt_ref, b2_ref,
             w3t_ref, b3_ref, o_ref):
        t = jnp.dot(xp_ref[...], w1t_ref[...],
                    preferred_element_type=jnp.float32, precision=hp)
        t = jnp.maximum(t + b1_ref[...], 0.0)
        acc2 = gi_ref[:, 0, :]
        for s in range(1, L):
            acc2 = acc2 + gi_ref[:, s, :]
        y = acc2 * (1.0 / L)
        h = (jnp.dot(y, w2at_ref[...], preferred_element_type=jnp.float32,
                     precision=hp)
             + jnp.dot(t, w2bt_ref[...], preferred_element_type=jnp.float32,
                       precision=hp))
        h = jnp.maximum(h + b2_ref[...], 0.0)
        o_ref[...] = (jnp.dot(h, w3t_ref[...], preferred_element_type=jnp.float32,
                              precision=hp)
                      + b3_ref[...])

    return pl.pallas_call(
        body,
        grid=(B // BB,),
        in_specs=[
            pl.BlockSpec((BB, F), lambda i: (i, 0)),
            pl.BlockSpec((BB, gi.shape[1], gi.shape[2]), lambda i: (i, 0, 0)),
            pl.BlockSpec(w1t.shape, lambda i: (0, 0)),
            pl.BlockSpec(b1.shape, lambda i: (0, 0)),
            pl.BlockSpec(w2at.shape, lambda i: (0, 0)),
            pl.BlockSpec(w2bt.shape, lambda i: (0, 0)),
            pl.BlockSpec(b2.shape, lambda i: (0, 0)),
            pl.BlockSpec(w3t.shape, lambda i: (0, 0)),
            pl.BlockSpec(b3.shape, lambda i: (0, 0)),
        ],
        out_specs=pl.BlockSpec((BB, w3t.shape[1]), lambda i: (i, 0)),
        out_shape=jax.ShapeDtypeStruct((B, w3t.shape[1]), jnp.float32),
    )(xp, gi, w1t, b1, w2at, w2bt, b2, w3t, b3)


def kernel(trigram_ids, interacted_rate, trigram_table, subreddit_table,
           W1, b1, W2, b2, W3, b3):
    B, S, TRI = trigram_ids.shape
    L = interacted_rate.shape[1]
    EMB = trigram_table.shape[1]

    LP = 64
    tri_idx = trigram_ids.astype(jnp.int32)
    ir32 = interacted_rate.astype(jnp.int32)
    int_pad = jnp.concatenate([ir32, ir32[:, :LP - L]], axis=1)
    int_idx = int_pad.reshape(-1, 64)

    g_pool, g_int = _sc_gather_both(trigram_table, tri_idx,
                                    subreddit_table, int_idx, S)

    xp = g_pool.reshape(B, TRI * EMB)
    gi = g_int.reshape(B, LP, EMB)

    return _tc_mlp(
        xp, gi, L,
        W1.T * (1.0 / S), b1.reshape(1, -1),
        W2[:, :EMB].T, W2[:, EMB:].T, b2.reshape(1, -1),
        W3.T, b3.reshape(1, -1),
    )


# speedup vs baseline: 2.5098x; 1.0708x over previous
import dataclasses
import functools

import jax
import jax.numpy as jnp
from jax import lax
from jax.experimental import pallas as pl
from jax.experimental.pallas import tpu as pltpu
from jax.experimental.pallas import tpu_sc as plsc

NC = 2
NS = 16
NW = NC * NS


K = 4
R = 64
CH = K * R
SBR = 16


def _emit_table_loop(table_hbm, idx_hbm, out_hbm, idx_w_base, out_w_base,
                     n_super, idx_v, rows_v, gsem, osem):
    n_chunks = n_super * 4

    def wait_out(b):
        pltpu.make_async_copy(rows_v[b], out_hbm.at[pl.ds(0, CH)], osem[b]).wait()

    def wait_gathers(b):
        pltpu.make_async_copy(out_hbm.at[pl.ds(0, CH)], rows_v[b], gsem[b]).wait()

    @pl.loop(0, n_super // 2)
    def _(gp):
        for ib in (0, 1):
            sb = gp * 2 + ib
            pltpu.sync_copy(idx_hbm.at[pl.ds(idx_w_base + sb * SBR, SBR)],
                            idx_v[ib])
            for c in range(4):
                b = c & 1
                if c >= 2:
                    wait_out(b)
                else:
                    @pl.when(sb >= 1)
                    def _w():
                        wait_out(b)
                for j in range(K):
                    pltpu.async_copy(
                        table_hbm.at[idx_v[ib].at[c * K + j]],
                        rows_v[b].at[pl.ds(j * R, R)],
                        gsem[b])
                prev_out = out_w_base + (sb * 4 + c - 1) * CH

                def _drain(prev_out=prev_out, b=b):
                    wait_gathers(1 - b)
                    pltpu.async_copy(rows_v[1 - b],
                                     out_hbm.at[pl.ds(prev_out, CH)],
                                     osem[1 - b])
                if c >= 1:
                    _drain()
                else:
                    @pl.when(sb >= 1)
                    def _d():
                        _drain()
    wait_gathers(1)
    pltpu.async_copy(
        rows_v[1],
        out_hbm.at[pl.ds(out_w_base + (n_chunks - 1) * CH, CH)], osem[1])
    wait_out(0)
    wait_out(1)


def _emit_pooled_loop(table_hbm, idx_hbm, out_hbm, wid, idx_v, idx_t, rows_v,
                      pooled_v, gsem, osem, isem, n_super, s_len):
    CKI = 5
    CROWS = CKI * R
    OROWS = CROWS // s_len
    out_base = wid * (n_super * 2 * R)
    TPB = s_len * R

    def fire_chunk(ibuf, c, b):
        for j in range(CKI):
            pltpu.async_copy(
                table_hbm.at[idx_t[ibuf].at[pl.ds((c * CKI + j) * R, R)]],
                rows_v[b].at[pl.ds(j * R, R)],
                gsem[b])

    def wait_gathers(b):
        pltpu.make_async_copy(table_hbm.at[pl.ds(0, CROWS)], rows_v[b],
                              gsem[b]).wait()

    def wait_out(pb):
        pltpu.make_async_copy(pooled_v[pb], out_hbm.at[pl.ds(0, OROWS)],
                              osem[pb]).wait()

    def load_idx_sync(sb, ibuf):
        b0 = wid * 2 * n_super + sb * 2
        pltpu.sync_copy(idx_hbm.at[b0], idx_v[ibuf].at[pl.ds(0, s_len)])
        pltpu.sync_copy(idx_hbm.at[b0 + 1],
                        idx_v[ibuf].at[pl.ds(s_len, s_len)])

    def load_idx_async(sb, ibuf):
        b0 = wid * 2 * n_super + sb * 2
        pltpu.async_copy(idx_hbm.at[b0], idx_v[ibuf].at[pl.ds(0, s_len)],
                         isem)
        pltpu.async_copy(idx_hbm.at[b0 + 1],
                         idx_v[ibuf].at[pl.ds(s_len, s_len)], isem)

    def wait_idx():
        for _ in range(2):
            pltpu.make_async_copy(idx_hbm.at[0],
                                  idx_v[0].at[pl.ds(0, s_len)], isem).wait()

    lane = jax.lax.iota(jnp.int32, 16)

    def transpose_idx(ibuf):
        for bi in (0, 1):
            for g in range(4):

                @pl.loop(0, s_len)
                def _(s):
                    v = idx_v[ibuf][bi * s_len + s, pl.ds(g * 16, 16)]
                    dst = (bi * TPB + g * 16 * s_len + s) + lane * s_len
                    plsc.store_scatter(idx_t[ibuf], [dst], v)

    def reduce_chunk(b, pb):
        @pl.loop(0, OROWS)
        def _(orow):
            row0 = orow * s_len

            @pl.loop(0, 8)
            def _(g):
                goff = g * 16
                a = rows_v[b][row0, pl.ds(goff, 16)]
                bacc = rows_v[b][row0 + 1, pl.ds(goff, 16)]
                for s in range(2, s_len, 2):
                    a = a + rows_v[b][row0 + s, pl.ds(goff, 16)]
                    bacc = bacc + rows_v[b][row0 + s + 1, pl.ds(goff, 16)]
                pooled_v[pb][orow, pl.ds(goff, 16)] = a + bacc

    load_idx_sync(0, 0)
    transpose_idx(0)
    fire_chunk(0, 0, 0)
    if n_super > 1:
        load_idx_async(1, 1)

    @pl.loop(0, n_super // 2)
    def _(gp):
        for ib in (0, 1):
            sb = gp * 2 + ib
            for c in range(8):
                b = c & 1
                ci = sb * 8 + c
                wait_gathers(b)
                if c == 0:
                    @pl.when(jnp.logical_and(sb >= 1, sb <= n_super - 2))
                    def _pf():
                        load_idx_async(sb + 1, 1 - ib)
                if c < 7:
                    fire_chunk(ib, c + 1, 1 - b)
                else:
                    @pl.when(sb <= n_super - 2)
                    def _nx():
                        wait_idx()
                        transpose_idx(1 - ib)
                        fire_chunk(1 - ib, 0, 1 - b)
                pb = c & 1
                if c >= 2:
                    wait_out(pb)
                else:
                    @pl.when(sb >= 1)
                    def _wo():
                        wait_out(pb)
                reduce_chunk(b, pb)
                pltpu.async_copy(pooled_v[pb],
                                 out_hbm.at[pl.ds(out_base + ci * OROWS,
                                                  OROWS)],
                                 osem[pb])
    wait_out(0)
    wait_out(1)


def _sc_gather_both(tri_table, tri_idx, int_table, int_idx, s_len):
    BT, _, TRI_ = tri_idx.shape
    n_int = int_idx.shape[0]
    int_pw = n_int // NW
    tri_ns = BT // NW // 2
    int_ns = int_pw // SBR
    assert BT % (NW * 2) == 0 and int_pw % SBR == 0 and int_ns % 2 == 0
    n_pool = BT * TRI_
    D = tri_table.shape[1]

    mesh = plsc.VectorSubcoreMesh(core_axis_name="c", subcore_axis_name="s")
    cp = pltpu.CompilerParams()
    if "needs_layout_passes" in pltpu.CompilerParams.__dataclass_fields__:
        cp = dataclasses.replace(cp, needs_layout_passes=False)

    @functools.partial(
        pl.kernel,
        mesh=mesh,
        compiler_params=cp,
        out_type=(jax.ShapeDtypeStruct((n_pool, D), jnp.float32),
                  jax.ShapeDtypeStruct((n_int * R, D), jnp.float32)),
        scratch_types=[
            pltpu.VMEM((2 * s_len, R), jnp.int32),
            pltpu.VMEM((2 * s_len, R), jnp.int32),
            pltpu.VMEM((SBR, R), jnp.int32),
            pltpu.VMEM((SBR, R), jnp.int32),
            pltpu.VMEM((2560,), jnp.int32),
            pltpu.VMEM((2560,), jnp.int32),
            pltpu.VMEM((320, 128), jnp.float32),
            pltpu.VMEM((320, 128), jnp.float32),
            pltpu.VMEM((16, 128), jnp.float32),
            pltpu.VMEM((16, 128), jnp.float32),
            pltpu.SemaphoreType.DMA,
            pltpu.SemaphoreType.DMA,
            pltpu.SemaphoreType.DMA,
            pltpu.SemaphoreType.DMA,
            pltpu.SemaphoreType.DMA,
        ],
    )
    def gather_kernel(tri_t_hbm, tri_i_hbm, int_t_hbm, int_i_hbm,
                      tri_o_hbm, int_o_hbm,
                      idx0, idx1, iidx0, iidx1, idxt0, idxt1,
                      rows0, rows1, pool0, pool1,
                      g0, g1, o0, o1, isem):
        wid = lax.axis_index("s") * NC + lax.axis_index("c")
        with jax.named_scope("tri_pooled_gather"):
            _emit_pooled_loop(tri_t_hbm, tri_i_hbm, tri_o_hbm, wid,
                              (idx0, idx1), (idxt0, idxt1),
                              (rows0, rows1), (pool0, pool1),
                              (g0, g1), (o0, o1), isem, tri_ns, s_len)
        with jax.named_scope("int_gather"):
            _emit_table_loop(int_t_hbm, int_i_hbm, int_o_hbm,
                             wid * int_pw, wid * int_pw * R, int_ns,
                             (iidx0, iidx1),
                             (rows0.at[pl.ds(0, CH)], rows1.at[pl.ds(0, CH)]),
                             (g0, g1), (o0, o1))

    return gather_kernel(tri_table, tri_idx, int_table, int_idx)


def _tc_mlp(xp, gi, true_l, tri, w1t, b1, w2at, w2bt, b2, w3t, b3):
    B = xp.shape[0] // tri
    F = tri * xp.shape[1]
    L = true_l
    BB = 256
    hp = jax.lax.Precision.HIGHEST

    def body(xp_ref, gi_ref, w1t_ref, b1_ref, w2at_ref, w2bt_ref, b2_ref,
             w3t_ref, b3_ref, o_ref):
        x = xp_ref[...].reshape(BB, F)
        t = jnp.dot(x, w1t_ref[...],
                    preferred_element_type=jnp.float32, precision=hp)
        t = jnp.maximum(t + b1_ref[...], 0.0)
        acc2 = gi_ref[:, 0, :]
        for s in range(1, L):
            acc2 = acc2 + gi_ref[:, s, :]
        y = acc2 * (1.0 / L)
        h = (jnp.dot(y, w2at_ref[...], preferred_element_type=jnp.float32,
                     precision=hp)
             + jnp.dot(t, w2bt_ref[...], preferred_element_type=jnp.float32,
                       precision=hp))
        h = jnp.maximum(h + b2_ref[...], 0.0)
        o_ref[...] = (jnp.dot(h, w3t_ref[...], preferred_element_type=jnp.float32,
                              precision=hp)
                      + b3_ref[...])

    return pl.pallas_call(
        body,
        grid=(B // BB,),
        in_specs=[
            pl.BlockSpec((BB * tri, xp.shape[1]), lambda i: (i, 0)),
            pl.BlockSpec((BB, gi.shape[1], gi.shape[2]), lambda i: (i, 0, 0)),
            pl.BlockSpec(w1t.shape, lambda i: (0, 0)),
            pl.BlockSpec(b1.shape, lambda i: (0, 0)),
            pl.BlockSpec(w2at.shape, lambda i: (0, 0)),
            pl.BlockSpec(w2bt.shape, lambda i: (0, 0)),
            pl.BlockSpec(b2.shape, lambda i: (0, 0)),
            pl.BlockSpec(w3t.shape, lambda i: (0, 0)),
            pl.BlockSpec(b3.shape, lambda i: (0, 0)),
        ],
        out_specs=pl.BlockSpec((BB, w3t.shape[1]), lambda i: (i, 0)),
        out_shape=jax.ShapeDtypeStruct((B, w3t.shape[1]), jnp.float32),
    )(xp, gi, w1t, b1, w2at, w2bt, b2, w3t, b3)


def kernel(trigram_ids, interacted_rate, trigram_table, subreddit_table,
           W1, b1, W2, b2, W3, b3):
    B, S, TRI = trigram_ids.shape
    L = interacted_rate.shape[1]
    EMB = trigram_table.shape[1]

    LP = 64
    tri_idx = trigram_ids.astype(jnp.int32)
    ir32 = interacted_rate.astype(jnp.int32)
    int_pad = jnp.concatenate([ir32, ir32[:, :LP - L]], axis=1)
    int_idx = int_pad.reshape(-1, 64)

    g_pool, g_int = _sc_gather_both(trigram_table, tri_idx,
                                    subreddit_table, int_idx, S)

    gi = g_int.reshape(B, LP, EMB)

    return _tc_mlp(
        g_pool, gi, L, TRI,
        W1.T * (1.0 / S), b1.reshape(1, -1),
        W2[:, :EMB].T, W2[:, EMB:].T, b2.reshape(1, -1),
        W3.T, b3.reshape(1, -1),
    )


# speedup vs baseline: 2.5132x; 1.0014x over previous
import dataclasses
import functools

import jax
import jax.numpy as jnp
from jax import lax
from jax.experimental import pallas as pl
from jax.experimental.pallas import tpu as pltpu
from jax.experimental.pallas import tpu_sc as plsc

NC = 2
NS = 16
NW = NC * NS


K = 4
R = 64
CH = K * R
SBR = 16


def _emit_table_loop(table_hbm, idx_hbm, out_hbm, idx_w_base, out_w_base,
                     n_super, idx_v, rows_v, gsem, osem):
    n_chunks = n_super * 4

    def wait_out(b):
        pltpu.make_async_copy(rows_v[b], out_hbm.at[pl.ds(0, CH)], osem[b]).wait()

    def wait_gathers(b):
        pltpu.make_async_copy(out_hbm.at[pl.ds(0, CH)], rows_v[b], gsem[b]).wait()

    @pl.loop(0, n_super // 2)
    def _(gp):
        for ib in (0, 1):
            sb = gp * 2 + ib
            pltpu.sync_copy(idx_hbm.at[pl.ds(idx_w_base + sb * SBR, SBR)],
                            idx_v[ib])
            for c in range(4):
                b = c & 1
                if c >= 2:
                    wait_out(b)
                else:
                    @pl.when(sb >= 1)
                    def _w():
                        wait_out(b)
                for j in range(K):
                    pltpu.async_copy(
                        table_hbm.at[idx_v[ib].at[c * K + j]],
                        rows_v[b].at[pl.ds(j * R, R)],
                        gsem[b])
                prev_out = out_w_base + (sb * 4 + c - 1) * CH

                def _drain(prev_out=prev_out, b=b):
                    wait_gathers(1 - b)
                    pltpu.async_copy(rows_v[1 - b],
                                     out_hbm.at[pl.ds(prev_out, CH)],
                                     osem[1 - b])
                if c >= 1:
                    _drain()
                else:
                    @pl.when(sb >= 1)
                    def _d():
                        _drain()
    wait_gathers(1)
    pltpu.async_copy(
        rows_v[1],
        out_hbm.at[pl.ds(out_w_base + (n_chunks - 1) * CH, CH)], osem[1])
    wait_out(0)
    wait_out(1)


def _emit_pooled_loop(table_hbm, idx_hbm, out_hbm, wid, idx_v, idx_t, rows_v,
                      pooled_v, gsem, osem, isem, n_super, s_len):
    CKI = 5
    CROWS = CKI * R
    OROWS = CROWS // s_len
    out_base = wid * (n_super * 2 * R)
    TPB = s_len * R

    def fire_chunk(ibuf, c, b):
        for j in range(CKI):
            pltpu.async_copy(
                table_hbm.at[idx_t[ibuf].at[pl.ds((c * CKI + j) * R, R)]],
                rows_v[b].at[pl.ds(j * R, R)],
                gsem[b])

    def wait_gathers(b):
        pltpu.make_async_copy(table_hbm.at[pl.ds(0, CROWS)], rows_v[b],
                              gsem[b]).wait()

    def wait_out(pb):
        pltpu.make_async_copy(pooled_v[pb], out_hbm.at[pl.ds(0, OROWS)],
                              osem[pb]).wait()

    def load_idx_sync(sb, ibuf):
        b0 = wid * 2 * n_super + sb * 2
        pltpu.sync_copy(idx_hbm.at[b0], idx_v[ibuf].at[pl.ds(0, s_len)])
        pltpu.sync_copy(idx_hbm.at[b0 + 1],
                        idx_v[ibuf].at[pl.ds(s_len, s_len)])

    def load_idx_async(sb, ibuf):
        b0 = wid * 2 * n_super + sb * 2
        pltpu.async_copy(idx_hbm.at[b0], idx_v[ibuf].at[pl.ds(0, s_len)],
                         isem)
        pltpu.async_copy(idx_hbm.at[b0 + 1],
                         idx_v[ibuf].at[pl.ds(s_len, s_len)], isem)

    def wait_idx():
        for _ in range(2):
            pltpu.make_async_copy(idx_hbm.at[0],
                                  idx_v[0].at[pl.ds(0, s_len)], isem).wait()

    lane = jax.lax.iota(jnp.int32, 16)

    def transpose_idx(ibuf):
        for bi in (0, 1):
            for g in range(4):

                @pl.loop(0, s_len)
                def _(s):
                    v = idx_v[ibuf][bi * s_len + s, pl.ds(g * 16, 16)]
                    dst = (bi * TPB + g * 16 * s_len + s) + lane * s_len
                    plsc.store_scatter(idx_t[ibuf], [dst], v)

    def reduce_chunk(b, pb):
        @pl.loop(0, OROWS)
        def _(orow):
            row0 = orow * s_len

            @pl.loop(0, 4)
            def _(g):
                for off in (0, 16):
                    goff = g * 32 + off
                    a = rows_v[b][row0, pl.ds(goff, 16)]
                    bacc = rows_v[b][row0 + 1, pl.ds(goff, 16)]
                    for s in range(2, s_len, 2):
                        a = a + rows_v[b][row0 + s, pl.ds(goff, 16)]
                        bacc = bacc + rows_v[b][row0 + s + 1, pl.ds(goff, 16)]
                    pooled_v[pb][orow, pl.ds(goff, 16)] = a + bacc

    load_idx_sync(0, 0)
    transpose_idx(0)
    fire_chunk(0, 0, 0)
    if n_super > 1:
        load_idx_async(1, 1)

    @pl.loop(0, n_super // 2)
    def _(gp):
        for ib in (0, 1):
            sb = gp * 2 + ib
            for c in range(8):
                b = c & 1
                ci = sb * 8 + c
                wait_gathers(b)
                if c == 0:
                    @pl.when(jnp.logical_and(sb >= 1, sb <= n_super - 2))
                    def _pf():
                        load_idx_async(sb + 1, 1 - ib)
                if c < 7:
                    fire_chunk(ib, c + 1, 1 - b)
                else:
                    @pl.when(sb <= n_super - 2)
                    def _nx():
                        wait_idx()
                        transpose_idx(1 - ib)
                        fire_chunk(1 - ib, 0, 1 - b)
                pb = c & 1
                if c >= 2:
                    wait_out(pb)
                else:
                    @pl.when(sb >= 1)
                    def _wo():
                        wait_out(pb)
                reduce_chunk(b, pb)
                pltpu.async_copy(pooled_v[pb],
                                 out_hbm.at[pl.ds(out_base + ci * OROWS,
                                                  OROWS)],
                                 osem[pb])
    wait_out(0)
    wait_out(1)


def _sc_gather_both(tri_table, tri_idx, int_table, int_idx, s_len):
    BT, _, TRI_ = tri_idx.shape
    n_int = int_idx.shape[0]
    int_pw = n_int // NW
    tri_ns = BT // NW // 2
    int_ns = int_pw // SBR
    assert BT % (NW * 2) == 0 and int_pw % SBR == 0 and int_ns % 2 == 0
    n_pool = BT * TRI_
    D = tri_table.shape[1]

    mesh = plsc.VectorSubcoreMesh(core_axis_name="c", subcore_axis_name="s")
    cp = pltpu.CompilerParams()
    if "needs_layout_passes" in pltpu.CompilerParams.__dataclass_fields__:
        cp = dataclasses.replace(cp, needs_layout_passes=False)

    @functools.partial(
        pl.kernel,
        mesh=mesh,
        compiler_params=cp,
        out_type=(jax.ShapeDtypeStruct((n_pool, D), jnp.float32),
                  jax.ShapeDtypeStruct((n_int * R, D), jnp.float32)),
        scratch_types=[
            pltpu.VMEM((2 * s_len, R), jnp.int32),
            pltpu.VMEM((2 * s_len, R), jnp.int32),
            pltpu.VMEM((SBR, R), jnp.int32),
            pltpu.VMEM((SBR, R), jnp.int32),
            pltpu.VMEM((2560,), jnp.int32),
            pltpu.VMEM((2560,), jnp.int32),
            pltpu.VMEM((320, 128), jnp.float32),
            pltpu.VMEM((320, 128), jnp.float32),
            pltpu.VMEM((16, 128), jnp.float32),
            pltpu.VMEM((16, 128), jnp.float32),
            pltpu.SemaphoreType.DMA,
            pltpu.SemaphoreType.DMA,
            pltpu.SemaphoreType.DMA,
            pltpu.SemaphoreType.DMA,
            pltpu.SemaphoreType.DMA,
        ],
    )
    def gather_kernel(tri_t_hbm, tri_i_hbm, int_t_hbm, int_i_hbm,
                      tri_o_hbm, int_o_hbm,
                      idx0, idx1, iidx0, iidx1, idxt0, idxt1,
                      rows0, rows1, pool0, pool1,
                      g0, g1, o0, o1, isem):
        wid = lax.axis_index("s") * NC + lax.axis_index("c")
        with jax.named_scope("tri_pooled_gather"):
            _emit_pooled_loop(tri_t_hbm, tri_i_hbm, tri_o_hbm, wid,
                              (idx0, idx1), (idxt0, idxt1),
                              (rows0, rows1), (pool0, pool1),
                              (g0, g1), (o0, o1), isem, tri_ns, s_len)
        with jax.named_scope("int_gather"):
            _emit_table_loop(int_t_hbm, int_i_hbm, int_o_hbm,
                             wid * int_pw, wid * int_pw * R, int_ns,
                             (iidx0, iidx1),
                             (rows0.at[pl.ds(0, CH)], rows1.at[pl.ds(0, CH)]),
                             (g0, g1), (o0, o1))

    return gather_kernel(tri_table, tri_idx, int_table, int_idx)


def _tc_mlp(xp, gi, true_l, tri, w1t, b1, w2at, w2bt, b2, w3t, b3):
    B = xp.shape[0] // tri
    F = tri * xp.shape[1]
    L = true_l
    BB = 128
    hp = jax.lax.Precision.HIGHEST

    def body(xp_ref, gi_ref, w1t_ref, b1_ref, w2at_ref, w2bt_ref, b2_ref,
             w3t_ref, b3_ref, o_ref):
        x = xp_ref[...].reshape(BB, F)
        t = jnp.dot(x, w1t_ref[...],
                    preferred_element_type=jnp.float32, precision=hp)
        t = jnp.maximum(t + b1_ref[...], 0.0)
        acc2 = gi_ref[:, 0, :]
        for s in range(1, L):
            acc2 = acc2 + gi_ref[:, s, :]
        y = acc2 * (1.0 / L)
        h = (jnp.dot(y, w2at_ref[...], preferred_element_type=jnp.float32,
                     precision=hp)
             + jnp.dot(t, w2bt_ref[...], preferred_element_type=jnp.float32,
                       precision=hp))
        h = jnp.maximum(h + b2_ref[...], 0.0)
        o_ref[...] = (jnp.dot(h, w3t_ref[...], preferred_element_type=jnp.float32,
                              precision=hp)
                      + b3_ref[...])

    return pl.pallas_call(
        body,
        grid=(B // BB,),
        in_specs=[
            pl.BlockSpec((BB * tri, xp.shape[1]), lambda i: (i, 0)),
            pl.BlockSpec((BB, gi.shape[1], gi.shape[2]), lambda i: (i, 0, 0)),
            pl.BlockSpec(w1t.shape, lambda i: (0, 0)),
            pl.BlockSpec(b1.shape, lambda i: (0, 0)),
            pl.BlockSpec(w2at.shape, lambda i: (0, 0)),
            pl.BlockSpec(w2bt.shape, lambda i: (0, 0)),
            pl.BlockSpec(b2.shape, lambda i: (0, 0)),
            pl.BlockSpec(w3t.shape, lambda i: (0, 0)),
            pl.BlockSpec(b3.shape, lambda i: (0, 0)),
        ],
        out_specs=pl.BlockSpec((BB, w3t.shape[1]), lambda i: (i, 0)),
        out_shape=jax.ShapeDtypeStruct((B, w3t.shape[1]), jnp.float32),
    )(xp, gi, w1t, b1, w2at, w2bt, b2, w3t, b3)


def kernel(trigram_ids, interacted_rate, trigram_table, subreddit_table,
           W1, b1, W2, b2, W3, b3):
    B, S, TRI = trigram_ids.shape
    L = interacted_rate.shape[1]
    EMB = trigram_table.shape[1]

    LP = 64
    tri_idx = trigram_ids.astype(jnp.int32)
    ir32 = interacted_rate.astype(jnp.int32)
    int_pad = jnp.concatenate([ir32, ir32[:, :LP - L]], axis=1)
    int_idx = int_pad.reshape(-1, 64)

    g_pool, g_int = _sc_gather_both(trigram_table, tri_idx,
                                    subreddit_table, int_idx, S)

    gi = g_int.reshape(B, LP, EMB)

    return _tc_mlp(
        g_pool, gi, L, TRI,
        W1.T * (1.0 / S), b1.reshape(1, -1),
        W2[:, :EMB].T, W2[:, EMB:].T, b2.reshape(1, -1),
        W3.T, b3.reshape(1, -1),
    )
